# Initial kernel scaffold; baseline (speedup 1.0000x reference)
import jax
import jax.numpy as jnp
from jax.experimental import pallas as pl


def kernel(x, codebook):
    raise NotImplementedError("write your pallas kernel here")


# speedup vs baseline: 1.2272x; 1.2272x over previous
import jax
import jax.numpy as jnp
from jax.experimental import pallas as pl
from jax.experimental.pallas import tpu as pltpu
from jax.experimental.pallas import tpu_sc as plsc

_BM = 256


def _dist_argmin_body(zsq_ref, x_ref, cbt_ref, csq_ref, idx_ref, min_ref):
    mm = jax.lax.dot_general(
        x_ref[...], cbt_ref[...], (((1,), (0,)), ((), ())),
        preferred_element_type=jnp.float32)
    d = (zsq_ref[...] - 2.0 * mm) + csq_ref[...]
    m = jnp.min(d, axis=1, keepdims=True)
    min_ref[...] = m
    k = d.shape[1]
    iota = jax.lax.broadcasted_iota(jnp.int32, d.shape, 1)
    idx_ref[...] = jnp.min(jnp.where(d == m, iota, k), axis=1, keepdims=True)


def _nearest_indices(flat, codebook):
    n, dim = flat.shape
    k = codebook.shape[0]
    zsq = jnp.sum(flat * flat, axis=1, keepdims=True)
    csq = jnp.sum(codebook * codebook, axis=1)[None, :]
    xb = flat.astype(jnp.bfloat16)
    cbt = codebook.T.astype(jnp.bfloat16)
    grid = n // _BM
    idx = pl.pallas_call(
        _dist_argmin_body,
        grid=(grid,),
        in_specs=[
            pl.BlockSpec((_BM, 1), lambda i: (i, 0)),
            pl.BlockSpec((_BM, dim), lambda i: (i, 0)),
            pl.BlockSpec((dim, k), lambda i: (0, 0)),
            pl.BlockSpec((1, k), lambda i: (0, 0)),
        ],
        out_specs=[pl.BlockSpec((_BM, 1), lambda i: (i, 0)),
                   pl.BlockSpec((_BM, 1), lambda i: (i, 0))],
        out_shape=[jax.ShapeDtypeStruct((n, 1), jnp.int32),
                   jax.ShapeDtypeStruct((n, 1), jnp.float32)],
    )(zsq, xb, cbt, csq)
    idx = idx[0]
    return idx.reshape(1, n)


_GATHER_WINDOW = 128


def _sc_gather_rows(codebook, indices):
    n = indices.shape[1]
    dim = codebook.shape[1]
    mesh = plsc.VectorSubcoreMesh(core_axis_name="c", subcore_axis_name="s")

    @pl.kernel(out_type=jax.ShapeDtypeStruct((n, dim), codebook.dtype),
               mesh=mesh)
    def kern(cb_hbm, i_hbm, o_hbm):
        def body(i_vmem, o_vmem):
            pltpu.sync_copy(cb_hbm.at[i_vmem.at[0]], o_vmem)

        pltpu.emit_pipeline(
            body,
            grid=(n // _GATHER_WINDOW,),
            in_specs=[pl.BlockSpec((1, _GATHER_WINDOW),
                                   index_map=lambda i: (0, i))],
            out_specs=[pl.BlockSpec((_GATHER_WINDOW, dim),
                                    index_map=lambda i: (i, 0))],
            core_axis_name=("c", "s"),
            dimension_semantics=(pltpu.PARALLEL,),
        )(i_hbm, o_hbm)

    return kern(codebook, indices)


def kernel(x, codebook):
    encoding = x
    dim = encoding.shape[-1]
    flat = encoding.reshape(-1, dim)
    idx = _nearest_indices(flat, codebook)
    q = _sc_gather_rows(codebook, idx).reshape(encoding.shape)
    return encoding + jax.lax.stop_gradient(q - encoding)
